# baseline (speedup 1.0000x reference)
import jax, jax.numpy as jnp
from jax.experimental import pallas as pl

def _bigin(x_ref, o_ref):
    o_ref[...] = x_ref[0:8, 0:128] * 2.0

@jax.jit
def kernel(attn_s):
    x = attn_s.reshape(8, 125000)
    t = pl.pallas_call(
        _bigin, out_shape=jax.ShapeDtypeStruct((8, 128), jnp.float32)
    )(x)
    return attn_s * t[0, 0]


# speedup vs baseline: 1.7352x; 1.7352x over previous
import jax
import jax.numpy as jnp
from jax import lax
from jax.experimental import pallas as pl
from jax.experimental.pallas import tpu as pltpu

_N = 1000000
_NP = 1000448
_C = _NP // 8
_K = 512
_SLABS = [(i * 16384, 16384) for i in range(7)] + [(114688, 10368)]


def _ukeys(x):
    b = lax.bitcast_convert_type(x, jnp.int32)
    ks = jnp.where(b < 0, jnp.bitwise_xor(b, jnp.int32(0x7FFFFFFF)), b)
    return lax.bitcast_convert_type(ks, jnp.uint32) ^ jnp.uint32(0x80000000)


def _u_to_f32(t):
    ts = lax.bitcast_convert_type(t ^ jnp.uint32(0x80000000), jnp.int32)
    db = jnp.where(ts < 0, jnp.bitwise_xor(ts, jnp.int32(0x7FFFFFFF)), ts)
    return lax.bitcast_convert_type(db, jnp.float32)


def _body(x_ref, o_ref, xc_ref, sems):
    for r in range(8):
        pltpu.make_async_copy(
            x_ref.at[0, pl.ds(r * _C, _C)], xc_ref.at[r], sems.at[r]
        ).start()
    for r in range(8):
        pltpu.make_async_copy(
            x_ref.at[0, pl.ds(r * _C, _C)], xc_ref.at[r], sems.at[r]
        ).wait()

    def slab(i):
        off, w = _SLABS[i]
        return xc_ref[:, off:off + w]

    def count_ge_f(cf):
        tot = jnp.int32(0)
        for i in range(8):
            tot += jnp.sum((slab(i) >= cf).astype(jnp.int32))
        return tot

    def count_ge_key(cand):
        tot = jnp.int32(0)
        for i in range(8):
            tot += jnp.sum((_ukeys(slab(i)) >= cand).astype(jnp.int32))
        return tot

    mx = jnp.float32(-jnp.inf)
    for i in range(8):
        mx = jnp.maximum(mx, jnp.max(slab(i)))
    umx = _ukeys(mx)

    p8 = umx & jnp.uint32(0xFF000000)
    cnt8 = count_ge_f(_u_to_f32(p8))
    ok8 = cnt8 >= _K
    t0 = jnp.where(ok8, p8, jnp.uint32(0))
    b0 = jnp.where(ok8, jnp.int32(23), jnp.int32(31))

    def cond(state):
        t, bitpos, cntt = state
        return (bitpos >= 0) & (cntt != _K)

    def body(state):
        t, bitpos, cntt = state
        cand = t | (jnp.uint32(1) << bitpos.astype(jnp.uint32))
        cnt = lax.cond(
            cand == jnp.uint32(0x80000000),
            lambda: count_ge_key(jnp.uint32(0x80000000)),
            lambda: count_ge_f(_u_to_f32(cand)),
        )
        take = cnt >= _K
        t = jnp.where(take, cand, t)
        cntt = jnp.where(take, cnt, cntt)
        return (t, bitpos - 1, cntt)

    t, _, cntt = lax.while_loop(
        cond, body, (t0, b0, jnp.int32(0x40000000))
    )

    def min_ge(c):
        mn = jnp.float32(jnp.inf)
        for i in range(8):
            xs = slab(i)
            mn = jnp.minimum(mn, jnp.min(jnp.where(xs >= c, xs, jnp.inf)))
        return mn

    delta = lax.cond(
        cntt == _K,
        lambda: min_ge(_u_to_f32(t)),
        lambda: _u_to_f32(t),
    )

    shift = mx - delta
    s = jnp.float32(0.0)
    for i in range(8):
        s += jnp.sum(jnp.maximum(slab(i) - shift, 0.0))
    inv = 1.0 / (s + jnp.float32(1e-7))
    o_ref[0] = shift
    o_ref[1] = inv


@jax.jit
def kernel(attn_s):
    xp = jnp.pad(attn_s, ((0, 0), (0, _NP - _N)), constant_values=-jnp.inf)
    si = pl.pallas_call(
        _body,
        out_specs=pl.BlockSpec(memory_space=pltpu.SMEM),
        out_shape=jax.ShapeDtypeStruct((2,), jnp.float32),
        scratch_shapes=[
            pltpu.VMEM((8, _C), jnp.float32),
            pltpu.SemaphoreType.DMA((8,)),
        ],
    )(xp)
    return jnp.maximum(attn_s - si[0], 0.0) * si[1]
